# baseline (speedup 1.0000x reference)
import functools

import jax
import jax.numpy as jnp
from jax.experimental import pallas as pl

_B = 128
_V = 100000
_ROWS_PER_BLOCK = 8
_NUM_BLOCKS = _B // _ROWS_PER_BLOCK


@functools.cache
def _gumbel_const():
    key = jax.random.key(42)
    u = jax.random.uniform(key, (_B, _V), dtype=jnp.float32,
                           minval=1e-20, maxval=1.0)
    return -jnp.log(-jnp.log(u))


def _argmax_body(logits_ref, gumbel_ref, out_ref):
    x = logits_ref[...] + gumbel_ref[...]
    m = jnp.max(x, axis=-1, keepdims=True)
    iota = jax.lax.broadcasted_iota(jnp.int32, x.shape, 1)
    idx = jnp.min(jnp.where(x == m, iota, _V), axis=-1)
    out_ref[0, 0, :] = idx


def _dense_argmax(logits, gumbel):
    out = pl.pallas_call(
        _argmax_body,
        grid=(_NUM_BLOCKS,),
        in_specs=[
            pl.BlockSpec((_ROWS_PER_BLOCK, _V), lambda i: (i, 0)),
            pl.BlockSpec((_ROWS_PER_BLOCK, _V), lambda i: (i, 0)),
        ],
        out_specs=pl.BlockSpec((1, 1, _ROWS_PER_BLOCK), lambda i: (i, 0, 0)),
        out_shape=jax.ShapeDtypeStruct((_NUM_BLOCKS, 1, _ROWS_PER_BLOCK),
                                       jnp.int32),
    )(logits, gumbel)
    return out.reshape(_B)


def kernel(logits):
    idx = _dense_argmax(logits, _gumbel_const())
    return idx.astype(jnp.int64)


# speedup vs baseline: 1.1706x; 1.1706x over previous
import functools

import numpy as np

import jax
import jax.numpy as jnp
from jax import lax
from jax.experimental import pallas as pl
from jax.experimental.pallas import tpu as pltpu
from jax.experimental.pallas import tpu_sc as plsc

_B = 128
_V = 100000
_K = 1024
_NW = 32
_R = _B // _NW
_L = 16
_C = 20000
_NCHUNK = _V // _C

_ROWS_PER_BLOCK = 8
_NUM_BLOCKS = _B // _ROWS_PER_BLOCK


@functools.cache
def _consts():
    with jax.ensure_compile_time_eval():
        return _consts_impl()


def _consts_impl():
    key = jax.random.key(42)
    u = jax.random.uniform(key, (_B, _V), dtype=jnp.float32,
                           minval=1e-20, maxval=1.0)
    gumbel = -jnp.log(-jnp.log(u))
    g = np.asarray(gumbel)
    topv, topi = jax.lax.top_k(gumbel, _K + 1)
    topv, topi = np.asarray(topv), np.asarray(topi)
    cand = np.sort(topi[:, :_K], axis=1)
    g_sub = topv[:, _K]

    maxcnt = 0
    for r in range(_B):
        maxcnt = max(maxcnt, int(np.bincount(cand[r] // _C,
                                             minlength=_NCHUNK).max()))
    P = int(((maxcnt + _L - 1) // _L) * _L)

    cand_off = np.zeros((_B, _NCHUNK, P), np.int32)
    cand_g = np.full((_B, _NCHUNK, P), -1e30, np.float32)
    cand_j = np.full((_B, _NCHUNK, P), _V, np.int32)
    for r in range(_B):
        for c in range(_NCHUNK):
            sel = cand[r][(cand[r] >= c * _C) & (cand[r] < (c + 1) * _C)]
            n = sel.size
            cand_off[r, c, :n] = sel - c * _C
            cand_g[r, c, :n] = g[r, sel]
            cand_j[r, c, :n] = sel
    gsub_splat = np.repeat(g_sub[:, None], _L, axis=1).astype(np.float32)
    return dict(gumbel=gumbel, P=P,
                cand_off=jnp.asarray(cand_off),
                cand_g=jnp.asarray(cand_g),
                cand_j=jnp.asarray(cand_j),
                gsub=jnp.asarray(gsub_splat))


def _make_sc(P):
    mesh = plsc.VectorSubcoreMesh(core_axis_name="c", subcore_axis_name="s")

    @functools.partial(
        pl.kernel,
        out_type=(jax.ShapeDtypeStruct((_NW, _R, _L), jnp.int32),
                  jax.ShapeDtypeStruct((_NW, _R, _L), jnp.int32)),
        mesh=mesh,
        compiler_params=pltpu.CompilerParams(use_tc_tiling_on_sc=False,
                                             needs_layout_passes=False),
        scratch_types=[
            pltpu.VMEM((_C,), jnp.float32),
            pltpu.VMEM((_C,), jnp.float32),
            pltpu.VMEM((_R, _NCHUNK, P), jnp.int32),
            pltpu.VMEM((_R, _NCHUNK, P), jnp.float32),
            pltpu.VMEM((_R, _NCHUNK, P), jnp.int32),
            pltpu.VMEM((_R, _L), jnp.float32),
            pltpu.VMEM((_R, _L), jnp.int32),
            pltpu.VMEM((_R, _L), jnp.int32),
            pltpu.SemaphoreType.DMA,
            pltpu.SemaphoreType.DMA,
        ],
    )
    def sc_fn(logits, cand_off, cand_g, cand_j, gsub,
              out_j, out_ok,
              buf0, buf1, off_v, g_v, j_v, gsub_v, rj_v, rok_v,
              sem0, sem1):
        wid = lax.axis_index("s") * 2 + lax.axis_index("c")
        base = wid * _R
        pltpu.sync_copy(cand_off.at[pl.ds(base, _R)], off_v)
        pltpu.sync_copy(cand_g.at[pl.ds(base, _R)], g_v)
        pltpu.sync_copy(cand_j.at[pl.ds(base, _R)], j_v)
        pltpu.sync_copy(gsub.at[pl.ds(base, _R)], gsub_v)
        bufs = (buf0, buf1)
        sems = (sem0, sem1)
        for r4 in range(_R):
            row = base + r4
            descs = [None] * _NCHUNK
            descs[0] = pltpu.async_copy(
                logits.at[row, pl.ds(0, _C)], bufs[0], sems[0])
            m_d = jnp.full((_L,), -jnp.inf, jnp.float32)
            m_c = jnp.full((_L,), -jnp.inf, jnp.float32)
            jx = jnp.full((_L,), _V, jnp.int32)
            for c in range(_NCHUNK):
                if c + 1 < _NCHUNK:
                    descs[c + 1] = pltpu.async_copy(
                        logits.at[row, pl.ds((c + 1) * _C, _C)],
                        bufs[(c + 1) % 2], sems[(c + 1) % 2])
                descs[c].wait()
                buf = bufs[c % 2]

                def dense_body(i, m, buf=buf):
                    return jnp.maximum(m, buf[pl.ds(i * _L, _L)])

                m_d = lax.fori_loop(0, _C // _L, dense_body, m_d)
                for p0 in range(0, P, _L):
                    off = off_v[r4, c, pl.ds(p0, _L)]
                    gg = g_v[r4, c, pl.ds(p0, _L)]
                    cj = j_v[r4, c, pl.ds(p0, _L)]
                    s = plsc.load_gather(buf, [off]) + gg
                    upd = s > m_c
                    m_c = jnp.where(upd, s, m_c)
                    jx = jnp.where(upd, cj, jx)
            m_row = jnp.max(m_d)
            mc_row = jnp.max(m_c)
            j_row = jnp.min(jnp.where(m_c == mc_row, jx, _V))
            thr = gsub_v[r4, :] + m_row
            rj_v[r4, :] = jnp.full((_L,), j_row, jnp.int32)
            rok_v[r4, :] = (mc_row > thr).astype(jnp.int32)
        pltpu.sync_copy(rj_v, out_j.at[wid])
        pltpu.sync_copy(rok_v, out_ok.at[wid])

    return sc_fn


def _argmax_body(logits_ref, gumbel_ref, out_ref):
    x = logits_ref[...] + gumbel_ref[...]
    m = jnp.max(x, axis=-1, keepdims=True)
    iota = jax.lax.broadcasted_iota(jnp.int32, x.shape, 1)
    idx = jnp.min(jnp.where(x == m, iota, _V), axis=-1)
    out_ref[0, 0, :] = idx


def _dense_argmax(logits, gumbel):
    out = pl.pallas_call(
        _argmax_body,
        grid=(_NUM_BLOCKS,),
        in_specs=[
            pl.BlockSpec((_ROWS_PER_BLOCK, _V), lambda i: (i, 0)),
            pl.BlockSpec((_ROWS_PER_BLOCK, _V), lambda i: (i, 0)),
        ],
        out_specs=pl.BlockSpec((1, 1, _ROWS_PER_BLOCK), lambda i: (i, 0, 0)),
        out_shape=jax.ShapeDtypeStruct((_NUM_BLOCKS, 1, _ROWS_PER_BLOCK),
                                       jnp.int32),
    )(logits, gumbel)
    return out.reshape(_B)


def kernel(logits):
    cs = _consts()
    sc_fn = _make_sc(cs["P"])
    out_j, out_ok = sc_fn(logits, cs["cand_off"], cs["cand_g"],
                          cs["cand_j"], cs["gsub"])
    j_fast = out_j[:, :, 0].reshape(_B)
    ok = jnp.all(out_ok[:, :, 0] == 1)
    idx = lax.cond(ok, lambda: j_fast,
                   lambda: _dense_argmax(logits, cs["gumbel"]))
    return idx.astype(jnp.int64)


# speedup vs baseline: 1.7095x; 1.4603x over previous
import functools

import numpy as np

import jax
import jax.numpy as jnp
from jax import lax
from jax.experimental import pallas as pl
from jax.experimental.pallas import tpu as pltpu
from jax.experimental.pallas import tpu_sc as plsc

_B = 128
_V = 100000
_K = 1024
_NW = 32
_R = _B // _NW
_L = 16
_C = 20000
_NCHUNK = _V // _C

_ROWS_PER_BLOCK = 8
_NUM_BLOCKS = _B // _ROWS_PER_BLOCK


@functools.cache
def _consts():
    with jax.ensure_compile_time_eval():
        return _consts_impl()


def _consts_impl():
    key = jax.random.key(42)
    u = jax.random.uniform(key, (_B, _V), dtype=jnp.float32,
                           minval=1e-20, maxval=1.0)
    gumbel = -jnp.log(-jnp.log(u))
    g = np.asarray(gumbel)
    topv, topi = jax.lax.top_k(gumbel, _K + 1)
    topv, topi = np.asarray(topv), np.asarray(topi)
    cand = np.sort(topi[:, :_K], axis=1)
    g_sub = topv[:, _K]

    maxcnt = 0
    for r in range(_B):
        maxcnt = max(maxcnt, int(np.bincount(cand[r] // _C,
                                             minlength=_NCHUNK).max()))
    P = int(((maxcnt + _L - 1) // _L) * _L)

    cand_off = np.zeros((_B, _NCHUNK, P), np.int32)
    cand_g = np.full((_B, _NCHUNK, P), -1e30, np.float32)
    cand_j = np.full((_B, _NCHUNK, P), _V, np.int32)
    for r in range(_B):
        for c in range(_NCHUNK):
            sel = cand[r][(cand[r] >= c * _C) & (cand[r] < (c + 1) * _C)]
            n = sel.size
            cand_off[r, c, :n] = sel - c * _C
            cand_g[r, c, :n] = g[r, sel]
            cand_j[r, c, :n] = sel
    gsub_splat = np.repeat(g_sub[:, None], _L, axis=1).astype(np.float32)
    return dict(gumbel=gumbel, P=P,
                cand_off=jnp.asarray(cand_off),
                cand_g=jnp.asarray(cand_g),
                cand_j=jnp.asarray(cand_j),
                gsub=jnp.asarray(gsub_splat))


def _make_sc(P):
    mesh = plsc.VectorSubcoreMesh(core_axis_name="c", subcore_axis_name="s")

    @functools.partial(
        pl.kernel,
        out_type=(jax.ShapeDtypeStruct((_NW, _R, _L), jnp.int32),
                  jax.ShapeDtypeStruct((_NW, _R, _L), jnp.int32)),
        mesh=mesh,
        compiler_params=pltpu.CompilerParams(use_tc_tiling_on_sc=False,
                                             needs_layout_passes=False),
        scratch_types=[
            pltpu.VMEM((_C,), jnp.float32),
            pltpu.VMEM((_C,), jnp.float32),
            pltpu.VMEM((_R, _NCHUNK, P), jnp.int32),
            pltpu.VMEM((_R, _NCHUNK, P), jnp.float32),
            pltpu.VMEM((_R, _NCHUNK, P), jnp.int32),
            pltpu.VMEM((_R, _L), jnp.float32),
            pltpu.VMEM((_R, _L), jnp.int32),
            pltpu.VMEM((_R, _L), jnp.int32),
            pltpu.SemaphoreType.DMA,
            pltpu.SemaphoreType.DMA,
        ],
    )
    def sc_fn(logits, cand_off, cand_g, cand_j, gsub,
              out_j, out_ok,
              buf0, buf1, off_v, g_v, j_v, gsub_v, rj_v, rok_v,
              sem0, sem1):
        wid = lax.axis_index("s") * 2 + lax.axis_index("c")
        base = wid * _R
        pltpu.sync_copy(cand_off.at[pl.ds(base, _R)], off_v)
        pltpu.sync_copy(cand_g.at[pl.ds(base, _R)], g_v)
        pltpu.sync_copy(cand_j.at[pl.ds(base, _R)], j_v)
        pltpu.sync_copy(gsub.at[pl.ds(base, _R)], gsub_v)
        bufs = (buf0, buf1)
        sems = (sem0, sem1)
        for r4 in range(_R):
            row = base + r4
            descs = [None] * _NCHUNK
            descs[0] = pltpu.async_copy(
                logits.at[row, pl.ds(0, _C)], bufs[0], sems[0])
            m_d = [jnp.full((_L,), -jnp.inf, jnp.float32) for _ in range(4)]
            m_c = jnp.full((_L,), -jnp.inf, jnp.float32)
            jx = jnp.full((_L,), _V, jnp.int32)
            for c in range(_NCHUNK):
                if c + 1 < _NCHUNK:
                    descs[c + 1] = pltpu.async_copy(
                        logits.at[row, pl.ds((c + 1) * _C, _C)],
                        bufs[(c + 1) % 2], sems[(c + 1) % 2])
                descs[c].wait()
                buf = bufs[c % 2]

                def dense_body(i, ms, buf=buf):
                    ms = list(ms)
                    for u in range(25):
                        x = buf[pl.ds((i * 25 + u) * _L, _L)]
                        ms[u % 4] = jnp.maximum(ms[u % 4], x)
                    return tuple(ms)

                m_d = list(lax.fori_loop(0, _C // (_L * 25), dense_body,
                                         tuple(m_d)))
                for p0 in range(0, P, _L):
                    off = off_v[r4, c, pl.ds(p0, _L)]
                    gg = g_v[r4, c, pl.ds(p0, _L)]
                    cj = j_v[r4, c, pl.ds(p0, _L)]
                    s = plsc.load_gather(buf, [off]) + gg
                    upd = s > m_c
                    m_c = jnp.where(upd, s, m_c)
                    jx = jnp.where(upd, cj, jx)
            m_row = jnp.max(jnp.maximum(jnp.maximum(m_d[0], m_d[1]),
                                        jnp.maximum(m_d[2], m_d[3])))
            mc_row = jnp.max(m_c)
            j_row = jnp.min(jnp.where(m_c == mc_row, jx, _V))
            thr = gsub_v[r4, :] + m_row
            rj_v[r4, :] = jnp.full((_L,), j_row, jnp.int32)
            rok_v[r4, :] = (mc_row > thr).astype(jnp.int32)
        pltpu.sync_copy(rj_v, out_j.at[wid])
        pltpu.sync_copy(rok_v, out_ok.at[wid])

    return sc_fn


def _argmax_body(logits_ref, gumbel_ref, out_ref):
    x = logits_ref[...] + gumbel_ref[...]
    m = jnp.max(x, axis=-1, keepdims=True)
    iota = jax.lax.broadcasted_iota(jnp.int32, x.shape, 1)
    idx = jnp.min(jnp.where(x == m, iota, _V), axis=-1)
    out_ref[0, 0, :] = idx


def _dense_argmax(logits, gumbel):
    out = pl.pallas_call(
        _argmax_body,
        grid=(_NUM_BLOCKS,),
        in_specs=[
            pl.BlockSpec((_ROWS_PER_BLOCK, _V), lambda i: (i, 0)),
            pl.BlockSpec((_ROWS_PER_BLOCK, _V), lambda i: (i, 0)),
        ],
        out_specs=pl.BlockSpec((1, 1, _ROWS_PER_BLOCK), lambda i: (i, 0, 0)),
        out_shape=jax.ShapeDtypeStruct((_NUM_BLOCKS, 1, _ROWS_PER_BLOCK),
                                       jnp.int32),
    )(logits, gumbel)
    return out.reshape(_B)


def kernel(logits):
    cs = _consts()
    sc_fn = _make_sc(cs["P"])
    out_j, out_ok = sc_fn(logits, cs["cand_off"], cs["cand_g"],
                          cs["cand_j"], cs["gsub"])
    j_fast = out_j[:, :, 0].reshape(_B)
    ok = jnp.all(out_ok[:, :, 0] == 1)
    idx = lax.cond(ok, lambda: j_fast,
                   lambda: _dense_argmax(logits, cs["gumbel"]))
    return idx.astype(jnp.int64)


# speedup vs baseline: 2.8445x; 1.6639x over previous
import functools

import numpy as np

import jax
import jax.numpy as jnp
from jax import lax
from jax.experimental import pallas as pl
from jax.experimental.pallas import tpu as pltpu
from jax.experimental.pallas import tpu_sc as plsc

_B = 128
_V = 100000
_K = 1024
_NW = 32
_R = _B // _NW
_L = 16

_ROWS_PER_BLOCK = 8
_NUM_BLOCKS = _B // _ROWS_PER_BLOCK


@functools.cache
def _consts():
    with jax.ensure_compile_time_eval():
        return _consts_impl()


def _consts_impl():
    key = jax.random.key(42)
    u = jax.random.uniform(key, (_B, _V), dtype=jnp.float32,
                           minval=1e-20, maxval=1.0)
    gumbel = -jnp.log(-jnp.log(u))
    g = np.asarray(gumbel)
    topv, topi = jax.lax.top_k(gumbel, _K + 1)
    topv, topi = np.asarray(topv), np.asarray(topi)
    cand = np.sort(topi[:, :_K], axis=1).astype(np.int32)
    g_sub = topv[:, _K]
    cand_g = np.take_along_axis(g, cand, axis=1).astype(np.float32)
    return dict(gumbel=gumbel,
                cand_cols=jnp.asarray(cand),
                cand_g=jnp.asarray(cand_g),
                gsub=jnp.asarray(g_sub.astype(np.float32)))


def _rowmax_body(logits_ref, out_ref):
    out_ref[0, 0, :] = jnp.max(logits_ref[...], axis=-1)


def _rowmax(logits):
    out = pl.pallas_call(
        _rowmax_body,
        grid=(_NUM_BLOCKS,),
        in_specs=[pl.BlockSpec((_ROWS_PER_BLOCK, _V), lambda i: (i, 0))],
        out_specs=pl.BlockSpec((1, 1, _ROWS_PER_BLOCK), lambda i: (i, 0, 0)),
        out_shape=jax.ShapeDtypeStruct((_NUM_BLOCKS, 1, _ROWS_PER_BLOCK),
                                       jnp.float32),
    )(logits)
    return out.reshape(_B)


@functools.cache
def _make_sc():
    mesh = plsc.VectorSubcoreMesh(core_axis_name="c", subcore_axis_name="s")

    @functools.partial(
        pl.kernel,
        out_type=(jax.ShapeDtypeStruct((_NW, _R, _L), jnp.float32),
                  jax.ShapeDtypeStruct((_NW, _R, _L), jnp.int32)),
        mesh=mesh,
        compiler_params=pltpu.CompilerParams(use_tc_tiling_on_sc=False,
                                             needs_layout_passes=False),
        scratch_types=[
            pltpu.VMEM((_R, _K), jnp.float32),
            pltpu.VMEM((_R, _K), jnp.float32),
            pltpu.VMEM((_R, _K), jnp.int32),
            pltpu.VMEM((_R, _L), jnp.float32),
            pltpu.VMEM((_R, _L), jnp.int32),
        ],
    )
    def sc_fn(cv, cg, cj, out_mc, out_j, cv_v, g_v, j_v, rmc_v, rj_v):
        wid = lax.axis_index("s") * 2 + lax.axis_index("c")
        base = wid * _R
        pltpu.sync_copy(cv.at[pl.ds(base, _R)], cv_v)
        pltpu.sync_copy(cg.at[pl.ds(base, _R)], g_v)
        pltpu.sync_copy(cj.at[pl.ds(base, _R)], j_v)
        for r4 in range(_R):
            m = jnp.full((_L,), -jnp.inf, jnp.float32)
            jx = jnp.full((_L,), _V, jnp.int32)
            for p0 in range(0, _K, _L):
                s = cv_v[r4, pl.ds(p0, _L)] + g_v[r4, pl.ds(p0, _L)]
                cjv = j_v[r4, pl.ds(p0, _L)]
                upd = s > m
                m = jnp.where(upd, s, m)
                jx = jnp.where(upd, cjv, jx)
            mc_row = jnp.max(m)
            j_row = jnp.min(jnp.where(m == mc_row, jx, _V))
            rmc_v[r4, :] = jnp.full((_L,), mc_row, jnp.float32)
            rj_v[r4, :] = jnp.full((_L,), j_row, jnp.int32)
        pltpu.sync_copy(rmc_v, out_mc.at[wid])
        pltpu.sync_copy(rj_v, out_j.at[wid])

    return sc_fn


def _argmax_body(logits_ref, gumbel_ref, out_ref):
    x = logits_ref[...] + gumbel_ref[...]
    m = jnp.max(x, axis=-1, keepdims=True)
    iota = jax.lax.broadcasted_iota(jnp.int32, x.shape, 1)
    idx = jnp.min(jnp.where(x == m, iota, _V), axis=-1)
    out_ref[0, 0, :] = idx


def _dense_argmax(logits, gumbel):
    out = pl.pallas_call(
        _argmax_body,
        grid=(_NUM_BLOCKS,),
        in_specs=[
            pl.BlockSpec((_ROWS_PER_BLOCK, _V), lambda i: (i, 0)),
            pl.BlockSpec((_ROWS_PER_BLOCK, _V), lambda i: (i, 0)),
        ],
        out_specs=pl.BlockSpec((1, 1, _ROWS_PER_BLOCK), lambda i: (i, 0, 0)),
        out_shape=jax.ShapeDtypeStruct((_NUM_BLOCKS, 1, _ROWS_PER_BLOCK),
                                       jnp.int32),
    )(logits, gumbel)
    return out.reshape(_B)


def kernel(logits):
    cs = _consts()
    m_row = _rowmax(logits)
    cv = jnp.take_along_axis(logits, cs["cand_cols"], axis=1)
    out_mc, out_j = _make_sc()(cv, cs["cand_g"], cs["cand_cols"])
    mc = out_mc[:, :, 0].reshape(_B)
    j_fast = out_j[:, :, 0].reshape(_B)
    ok = jnp.all(mc > m_row + cs["gsub"])
    idx = lax.cond(ok, lambda: j_fast,
                   lambda: _dense_argmax(logits, cs["gumbel"]))
    return idx.astype(jnp.int64)
